# baseline (speedup 1.0000x reference)
import jax
import jax.numpy as jnp
from jax.experimental import pallas as pl
from jax.experimental.pallas import tpu as pltpu

_N = 4
_P = 4096
_C8 = 8
_IBLK = 512
_IB = _P // _IBLK
_LMBDA = 1.0


def _chamfer_kernel(xts_ref, yt_ref, lik_ref, cham_ref, bits_ref,
                    colmin_ref, acc_ref):
    n = pl.program_id(0)
    i = pl.program_id(1)

    @pl.when(jnp.logical_and(n == 0, i == 0))
    def _():
        acc_ref[1] = 0.0
        acc_ref[2] = 0.0

    @pl.when(i == 0)
    def _():
        acc_ref[0] = 0.0
        colmin_ref[...] = jnp.full_like(colmin_ref[...], jnp.inf)
        acc_ref[2] += jnp.sum(jnp.log2(lik_ref[0]))

    xts = xts_ref[0]
    yt = yt_ref[0]
    x2 = 0.25 * jnp.sum(xts * xts, axis=0)
    y2 = jnp.sum(yt * yt, axis=0)
    rx = jax.lax.broadcasted_iota(jnp.int32, (_C8, _IBLK), 0)
    ry = jax.lax.broadcasted_iota(jnp.int32, (_C8, _P), 0)
    xa = jnp.where(rx == 3, 1.0, xts)
    ya = jnp.where(ry == 3, y2[None, :], yt)
    dp = jax.lax.dot_general(
        xa, ya, (((0,), (0,)), ((), ())),
        preferred_element_type=jnp.float32)
    acc_ref[0] += jnp.sum(jnp.maximum(jnp.min(dp, axis=1) + x2, 0.0))
    colmin_ref[0] = jnp.minimum(colmin_ref[0],
                                jnp.min(dp + x2[:, None], axis=0))

    @pl.when(i == _IB - 1)
    def _():
        acc_ref[1] += (acc_ref[0]
                       + jnp.sum(jnp.maximum(colmin_ref[0], 0.0))) / _P

    @pl.when(jnp.logical_and(n == _N - 1, i == _IB - 1))
    def _():
        cham_ref[0] = jnp.full((8, 128), acc_ref[1], jnp.float32)
        bits_ref[0] = jnp.full((8, 128), acc_ref[2], jnp.float32)


def _run(x_hat, likelihood_y, points, interpret=False):
    xts = (-2.0 * jnp.pad(x_hat, ((0, 0), (0, 0), (0, _C8 - 3)))
           ).transpose(0, 2, 1)
    yt = jnp.pad(points, ((0, 0), (0, 0), (0, _C8 - 3))).transpose(0, 2, 1)
    lik = likelihood_y.reshape(_N, 64, 128)
    cham, bits = pl.pallas_call(
        _chamfer_kernel,
        grid=(_N, _IB),
        in_specs=[
            pl.BlockSpec((1, _C8, _IBLK), lambda n, i: (n, 0, i)),
            pl.BlockSpec((1, _C8, _P), lambda n, i: (n, 0, 0)),
            pl.BlockSpec((1, 64, 128), lambda n, i: (n, 0, 0)),
        ],
        out_specs=[
            pl.BlockSpec((1, 8, 128), lambda n, i: (0, 0, 0)),
            pl.BlockSpec((1, 8, 128), lambda n, i: (0, 0, 0)),
        ],
        out_shape=[
            jax.ShapeDtypeStruct((1, 8, 128), jnp.float32),
            jax.ShapeDtypeStruct((1, 8, 128), jnp.float32),
        ],
        scratch_shapes=[
            pltpu.VMEM((1, _P), jnp.float32),
            pltpu.SMEM((3,), jnp.float32),
        ],
        interpret=interpret,
    )(xts, yt, lik)

    rec_loss = cham[0, 0, 0] / _N
    bit_y_loss = bits[0, 0, 0] / (-_N)
    bpp_y_loss = bit_y_loss / _P
    bit_loss = bit_y_loss
    bpp_loss = bit_loss / _P
    loss = bpp_loss + _LMBDA * rec_loss
    return (loss, bit_y_loss, bpp_y_loss, bit_loss, bpp_loss, rec_loss)


@jax.jit
def kernel(x_hat, likelihood_y, points):
    return _run(x_hat, likelihood_y, points)


# speedup vs baseline: 1.2523x; 1.2523x over previous
import jax
import jax.numpy as jnp
from jax.experimental import pallas as pl
from jax.experimental.pallas import tpu as pltpu

_N = 4
_P = 4096
_C8 = 8
_IBLK = 512
_IB = _P // _IBLK
_LMBDA = 1.0


def _chamfer_kernel(xts_ref, yt_ref, lik_ref, cham_ref, bits_ref,
                    colmin_ref, acc_ref):
    i = pl.program_id(1)

    @pl.when(i == 0)
    def _():
        acc_ref[0] = 0.0
        colmin_ref[...] = jnp.full_like(colmin_ref[...], jnp.inf)
        bits_ref[0] = jnp.full((8, 128), jnp.sum(jnp.log2(lik_ref[0])),
                               jnp.float32)

    xts = xts_ref[0]
    yt = yt_ref[0]
    x2 = 0.25 * jnp.sum(xts * xts, axis=0)[:, None]
    y2 = jnp.sum(yt * yt, axis=0)[None, :]
    nxy2 = jax.lax.dot_general(
        xts, yt, (((0,), (0,)), ((), ())),
        preferred_element_type=jnp.float32)
    d = (x2 + y2) + nxy2
    acc_ref[0] += jnp.sum(jnp.maximum(jnp.min(d, axis=1), 0.0))
    colmin_ref[0] = jnp.minimum(colmin_ref[0], jnp.min(d, axis=0))

    @pl.when(i == _IB - 1)
    def _():
        cham = (acc_ref[0]
                + jnp.sum(jnp.maximum(colmin_ref[0], 0.0))) / _P
        cham_ref[0] = jnp.full((8, 128), cham, jnp.float32)


def _run(x_hat, likelihood_y, points, interpret=False):
    xts = (-2.0 * jnp.pad(x_hat, ((0, 0), (0, 0), (0, _C8 - 3)))
           ).transpose(0, 2, 1)
    yt = jnp.pad(points, ((0, 0), (0, 0), (0, _C8 - 3))).transpose(0, 2, 1)
    lik = likelihood_y.reshape(_N, 64, 128)
    cham, bits = pl.pallas_call(
        _chamfer_kernel,
        grid=(_N, _IB),
        in_specs=[
            pl.BlockSpec((1, _C8, _IBLK), lambda n, i: (n, 0, i)),
            pl.BlockSpec((1, _C8, _P), lambda n, i: (n, 0, 0)),
            pl.BlockSpec((1, 64, 128), lambda n, i: (n, 0, 0)),
        ],
        out_specs=[
            pl.BlockSpec((1, 8, 128), lambda n, i: (n, 0, 0)),
            pl.BlockSpec((1, 8, 128), lambda n, i: (n, 0, 0)),
        ],
        out_shape=[
            jax.ShapeDtypeStruct((_N, 8, 128), jnp.float32),
            jax.ShapeDtypeStruct((_N, 8, 128), jnp.float32),
        ],
        scratch_shapes=[
            pltpu.VMEM((1, _P), jnp.float32),
            pltpu.SMEM((1,), jnp.float32),
        ],
        compiler_params=pltpu.CompilerParams(
            dimension_semantics=("parallel", "arbitrary")),
        interpret=interpret,
    )(xts, yt, lik)

    rec_loss = jnp.mean(cham[:, 0, 0])
    bit_y_loss = jnp.sum(bits[:, 0, 0]) / (-_N)
    bpp_y_loss = bit_y_loss / _P
    bit_loss = bit_y_loss
    bpp_loss = bit_loss / _P
    loss = bpp_loss + _LMBDA * rec_loss
    return (loss, bit_y_loss, bpp_y_loss, bit_loss, bpp_loss, rec_loss)


@jax.jit
def kernel(x_hat, likelihood_y, points):
    return _run(x_hat, likelihood_y, points)


# speedup vs baseline: 1.3523x; 1.0799x over previous
import jax
import jax.numpy as jnp
from jax.experimental import pallas as pl
from jax.experimental.pallas import tpu as pltpu

_N = 4
_P = 4096
_C8 = 8
_IBLK = 1024
_IB = _P // _IBLK
_LMBDA = 1.0


def _chamfer_kernel(xts_ref, yt_ref, lik_ref, cham_ref, bits_ref,
                    colmin_ref, acc_ref):
    i = pl.program_id(1)

    @pl.when(i == 0)
    def _():
        acc_ref[0] = 0.0
        colmin_ref[...] = jnp.full_like(colmin_ref[...], jnp.inf)
        bits_ref[0] = jnp.full((8, 128), jnp.sum(jnp.log2(lik_ref[0])),
                               jnp.float32)

    xts = xts_ref[0]
    yt = yt_ref[0]
    x2 = 0.25 * jnp.sum(xts * xts, axis=0)[:, None]
    y2 = jnp.sum(yt * yt, axis=0)[None, :]
    nxy2 = jax.lax.dot_general(
        xts, yt, (((0,), (0,)), ((), ())),
        preferred_element_type=jnp.float32)
    d = (x2 + y2) + nxy2
    acc_ref[0] += jnp.sum(jnp.maximum(jnp.min(d, axis=1), 0.0))
    colmin_ref[0] = jnp.minimum(colmin_ref[0], jnp.min(d, axis=0))

    @pl.when(i == _IB - 1)
    def _():
        cham = (acc_ref[0]
                + jnp.sum(jnp.maximum(colmin_ref[0], 0.0))) / _P
        cham_ref[0] = jnp.full((8, 128), cham, jnp.float32)


def _run(x_hat, likelihood_y, points, interpret=False):
    xts = (-2.0 * jnp.pad(x_hat, ((0, 0), (0, 0), (0, _C8 - 3)))
           ).transpose(0, 2, 1)
    yt = jnp.pad(points, ((0, 0), (0, 0), (0, _C8 - 3))).transpose(0, 2, 1)
    lik = likelihood_y.reshape(_N, 64, 128)
    cham, bits = pl.pallas_call(
        _chamfer_kernel,
        grid=(_N, _IB),
        in_specs=[
            pl.BlockSpec((1, _C8, _IBLK), lambda n, i: (n, 0, i)),
            pl.BlockSpec((1, _C8, _P), lambda n, i: (n, 0, 0)),
            pl.BlockSpec((1, 64, 128), lambda n, i: (n, 0, 0)),
        ],
        out_specs=[
            pl.BlockSpec((1, 8, 128), lambda n, i: (n, 0, 0)),
            pl.BlockSpec((1, 8, 128), lambda n, i: (n, 0, 0)),
        ],
        out_shape=[
            jax.ShapeDtypeStruct((_N, 8, 128), jnp.float32),
            jax.ShapeDtypeStruct((_N, 8, 128), jnp.float32),
        ],
        scratch_shapes=[
            pltpu.VMEM((1, _P), jnp.float32),
            pltpu.SMEM((1,), jnp.float32),
        ],
        compiler_params=pltpu.CompilerParams(
            dimension_semantics=("parallel", "arbitrary")),
        interpret=interpret,
    )(xts, yt, lik)

    rec_loss = jnp.mean(cham[:, 0, 0])
    bit_y_loss = jnp.sum(bits[:, 0, 0]) / (-_N)
    bpp_y_loss = bit_y_loss / _P
    bit_loss = bit_y_loss
    bpp_loss = bit_loss / _P
    loss = bpp_loss + _LMBDA * rec_loss
    return (loss, bit_y_loss, bpp_y_loss, bit_loss, bpp_loss, rec_loss)


@jax.jit
def kernel(x_hat, likelihood_y, points):
    return _run(x_hat, likelihood_y, points)


# speedup vs baseline: 1.4250x; 1.0538x over previous
import jax
import jax.numpy as jnp
from jax.experimental import pallas as pl
from jax.experimental.pallas import tpu as pltpu

_N = 4
_P = 4096
_C8 = 8
_IBLK = 2048
_IB = _P // _IBLK
_LMBDA = 1.0


def _chamfer_kernel(xts_ref, yt_ref, lik_ref, cham_ref, bits_ref,
                    colmin_ref, acc_ref):
    i = pl.program_id(1)

    @pl.when(i == 0)
    def _():
        acc_ref[0] = 0.0
        colmin_ref[...] = jnp.full_like(colmin_ref[...], jnp.inf)
        bits_ref[0] = jnp.full((8, 128), jnp.sum(jnp.log2(lik_ref[0])),
                               jnp.float32)

    xts = xts_ref[0]
    yt = yt_ref[0]
    x2 = 0.25 * jnp.sum(xts * xts, axis=0)[:, None]
    y2 = jnp.sum(yt * yt, axis=0)[None, :]
    nxy2 = jax.lax.dot_general(
        xts, yt, (((0,), (0,)), ((), ())),
        preferred_element_type=jnp.float32)
    d = (x2 + y2) + nxy2
    acc_ref[0] += jnp.sum(jnp.maximum(jnp.min(d, axis=1), 0.0))
    colmin_ref[0] = jnp.minimum(colmin_ref[0], jnp.min(d, axis=0))

    @pl.when(i == _IB - 1)
    def _():
        cham = (acc_ref[0]
                + jnp.sum(jnp.maximum(colmin_ref[0], 0.0))) / _P
        cham_ref[0] = jnp.full((8, 128), cham, jnp.float32)


def _run(x_hat, likelihood_y, points, interpret=False):
    xts = (-2.0 * jnp.pad(x_hat, ((0, 0), (0, 0), (0, _C8 - 3)))
           ).transpose(0, 2, 1)
    yt = jnp.pad(points, ((0, 0), (0, 0), (0, _C8 - 3))).transpose(0, 2, 1)
    lik = likelihood_y.reshape(_N, 64, 128)
    cham, bits = pl.pallas_call(
        _chamfer_kernel,
        grid=(_N, _IB),
        in_specs=[
            pl.BlockSpec((1, _C8, _IBLK), lambda n, i: (n, 0, i)),
            pl.BlockSpec((1, _C8, _P), lambda n, i: (n, 0, 0)),
            pl.BlockSpec((1, 64, 128), lambda n, i: (n, 0, 0)),
        ],
        out_specs=[
            pl.BlockSpec((1, 8, 128), lambda n, i: (n, 0, 0)),
            pl.BlockSpec((1, 8, 128), lambda n, i: (n, 0, 0)),
        ],
        out_shape=[
            jax.ShapeDtypeStruct((_N, 8, 128), jnp.float32),
            jax.ShapeDtypeStruct((_N, 8, 128), jnp.float32),
        ],
        scratch_shapes=[
            pltpu.VMEM((1, _P), jnp.float32),
            pltpu.SMEM((1,), jnp.float32),
        ],
        compiler_params=pltpu.CompilerParams(
            dimension_semantics=("parallel", "arbitrary")),
        interpret=interpret,
    )(xts, yt, lik)

    rec_loss = jnp.mean(cham[:, 0, 0])
    bit_y_loss = jnp.sum(bits[:, 0, 0]) / (-_N)
    bpp_y_loss = bit_y_loss / _P
    bit_loss = bit_y_loss
    bpp_loss = bit_loss / _P
    loss = bpp_loss + _LMBDA * rec_loss
    return (loss, bit_y_loss, bpp_y_loss, bit_loss, bpp_loss, rec_loss)


@jax.jit
def kernel(x_hat, likelihood_y, points):
    return _run(x_hat, likelihood_y, points)
